# baseline (speedup 1.0000x reference)
import functools

import jax
import jax.numpy as jnp
from jax import lax
from jax.experimental import pallas as pl
from jax.experimental.pallas import tpu as pltpu
from jax.experimental.pallas import tpu_sc as plsc

E = 16
D = 1024
DFF = 1024
T = 8192

BM = 512
NT = T // BM
G = NT + E
TP = G * BM

MW = 640
EGW = 128
OFFP_L = 128
USED_L = 160
SX_L = 192
SY_L = 320
PO_L = 448

RT = 1024
NRT = T // RT

NC = 2
NS = 16
NW = NC * NS
CH = T // NW
NRW = RT // CH
SUB = 64
NSUB = CH // SUB
SUBC = 32
NSUBC = CH // SUBC


def _gate_body(x_ref, wg_ref, bg_ref, er_ref, meta_ref, xb_ref, run_ref):
    t = pl.program_id(0)

    @pl.when(t == 0)
    def _():
        run_ref[...] = jnp.zeros_like(run_ref)

    xb = x_ref[...].astype(jnp.bfloat16)
    lo = lax.bitcast_convert_type(xb[:, :D // 2], jnp.uint16)
    hi = lax.bitcast_convert_type(xb[:, D // 2:], jnp.uint16)
    xb_ref[...] = lax.bitcast_convert_type(
        lo.astype(jnp.uint32) | (hi.astype(jnp.uint32) << 16), jnp.int32)
    logits = jnp.dot(x_ref[...], wg_ref[...],
                     preferred_element_type=jnp.float32) + bg_ref[...]
    m = jnp.max(logits, axis=1, keepdims=True)
    ii = lax.broadcasted_iota(jnp.int32, (RT, E), 1)
    eid = jnp.min(jnp.where(logits == m, ii, E), axis=1)
    oh = (ii == eid[:, None]).astype(jnp.float32)
    ri = lax.broadcasted_iota(jnp.int32, (RT, RT), 0)
    ci = lax.broadcasted_iota(jnp.int32, (RT, RT), 1)
    lt = (ci < ri).astype(jnp.bfloat16)
    within = jnp.dot(lt, oh.astype(jnp.bfloat16),
                     preferred_element_type=jnp.float32)
    run = run_ref[...]
    rank = jnp.sum(oh * (within + run), axis=1)
    er_ref[...] = ((eid << 16) | rank.astype(jnp.int32))[None, None, :]
    new_run = run + jnp.sum(oh, axis=0, keepdims=True)
    run_ref[...] = new_run

    @pl.when(t == NRT - 1)
    def _():
        tiles = jnp.ceil(new_run * (1.0 / BM))
        ut = (lax.broadcasted_iota(jnp.int32, (E, E), 0)
              <= lax.broadcasted_iota(jnp.int32, (E, E), 1)).astype(jnp.bfloat16)
        ends = jnp.dot(tiles.astype(jnp.bfloat16), ut,
                       preferred_element_type=jnp.float32)
        offp = ((ends - tiles) * BM).astype(jnp.int32)
        used = jnp.max(ends, axis=1, keepdims=True).astype(jnp.int32)
        gi = lax.broadcasted_iota(jnp.int32, (EGW, E), 0)
        endsi = ends.astype(jnp.int32)
        eg = jnp.minimum(
            jnp.sum((gi >= endsi).astype(jnp.int32), axis=1), E - 1)

        present = (tiles > 0).astype(jnp.bfloat16)
        ordi = jnp.dot(present, ut, preferred_element_type=jnp.float32)
        ordm1 = ordi.astype(jnp.int32) - 1
        pres_i = (tiles > 0).astype(jnp.int32)
        npres = jnp.sum(pres_i, axis=1, keepdims=True)
        o = jnp.sum(((gi >= endsi) & (pres_i > 0)).astype(jnp.int32), axis=1)
        o = jnp.minimum(o, jnp.maximum(npres[0, 0] - 1, 0))
        par = o & 1
        sxt = o + par
        syt = o + 1 - par
        last = npres[0, 0] - 1
        sxt = jnp.where(sxt > last, jnp.maximum(sxt - 2, 0), sxt)
        syt = jnp.where(syt > last, jnp.maximum(syt - 2, 0), syt)
        ei = lax.broadcasted_iota(jnp.int32, (EGW, E), 1)
        omask = (ordm1 == sxt[:, None]) & (pres_i > 0)
        sx = jnp.sum(jnp.where(omask, ei, 0), axis=1)
        omask = (ordm1 == syt[:, None]) & (pres_i > 0)
        sy = jnp.sum(jnp.where(omask, ei, 0), axis=1)
        meta_ref[...] = jnp.concatenate(
            [eg[None, :], offp, jnp.broadcast_to(used, (1, 48)),
             sx[None, :], sy[None, :], par[None, :],
             jnp.zeros((1, MW - PO_L - EGW), jnp.int32)], axis=1)


_gate_call = pl.pallas_call(
    _gate_body,
    grid=(NRT,),
    in_specs=[
        pl.BlockSpec((RT, D), lambda t: (t, 0)),
        pl.BlockSpec((D, E), lambda t: (0, 0)),
        pl.BlockSpec((1, E), lambda t: (0, 0)),
    ],
    out_specs=[
        pl.BlockSpec((1, 1, RT), lambda t: (t, 0, 0)),
        pl.BlockSpec((1, MW), lambda t: (0, 0)),
        pl.BlockSpec((RT, D // 2), lambda t: (t, 0)),
    ],
    out_shape=[
        jax.ShapeDtypeStruct((NRT, 1, RT), jnp.int32),
        jax.ShapeDtypeStruct((1, MW), jnp.int32),
        jax.ShapeDtypeStruct((T, D // 2), jnp.int32),
    ],
    scratch_shapes=[pltpu.VMEM((1, E), jnp.float32)],
    compiler_params=pltpu.CompilerParams(
        dimension_semantics=("arbitrary",)),
)


_sc_mesh = plsc.VectorSubcoreMesh(core_axis_name="c", subcore_axis_name="s")


@functools.partial(
    pl.kernel,
    out_type=[
        jax.ShapeDtypeStruct((TP, D // 2), jnp.int32),
        jax.ShapeDtypeStruct((NW, NSUB, SUB), jnp.int32),
    ],
    mesh=_sc_mesh,
    scratch_types=[
        pltpu.VMEM((CH,), jnp.int32),
        pltpu.VMEM((E,), jnp.int32),
        pltpu.VMEM((NSUB, SUB), jnp.int32),
        pltpu.VMEM((SUB, D // 2), jnp.int32),
        pltpu.VMEM((SUB, D // 2), jnp.int32),
        pltpu.SemaphoreType.DMA,
        pltpu.SemaphoreType.DMA,
    ],
    compiler_params=pltpu.CompilerParams(needs_layout_passes=False),
)
def _dispatch_kernel(x_hbm, er_hbm, meta_hbm, xsp_hbm, pos_hbm,
                     er_v, off_v, idx_v, rows0, rows1, rsem, ssem):
    w = lax.axis_index("s") * NC + lax.axis_index("c")
    base = w * CH
    r, lo = w // NRW, (w % NRW) * CH
    rows = (rows0, rows1)
    reads = [pltpu.async_copy(x_hbm.at[pl.ds(base, SUB)], rows0, rsem)]
    pltpu.sync_copy(meta_hbm.at[0, pl.ds(OFFP_L, E)], off_v)
    pltpu.sync_copy(er_hbm.at[r, 0, pl.ds(lo, CH)], er_v)
    for j in range(NSUB):
        for k in range(SUB // 16):
            o = j * SUB + k * 16
            ev = er_v[pl.ds(o, 16)]
            idx_v[j, pl.ds(k * 16, 16)] = (
                plsc.load_gather(off_v, [ev >> 16]) + (ev & 0xFFFF))
    scats = []
    for j in range(NSUB):
        reads[j].wait()
        scats.append(
            pltpu.async_copy(rows[j % 2], xsp_hbm.at[idx_v.at[j]], ssem))
        if j + 1 < NSUB:
            if j >= 1:
                scats[j - 1].wait()
            reads.append(pltpu.async_copy(
                x_hbm.at[pl.ds(base + (j + 1) * SUB, SUB)],
                rows[(j + 1) % 2], rsem))
    pltpu.sync_copy(idx_v, pos_hbm.at[w])
    for j in range(max(NSUB - 2, 0), NSUB):
        scats[j].wait()


def _gg(g, s):
    return jnp.minimum(g, s[0, USED_L] - 1)


def _ffn_body(s_ref, x_ref, w1x_ref, w1y_ref, b1_ref, w2x_ref, w2y_ref,
              b2_ref, o_ref, w1b_s, w2b_s, last_s):
    g = pl.program_id(0)

    @pl.when(g < s_ref[0, USED_L])
    def _():
        e = s_ref[0, g]

        @pl.when(jnp.logical_or(g == 0, e != last_s[0]))
        def _():
            p = s_ref[0, PO_L + g]

            @pl.when(p == 0)
            def _():
                w1b_s[...] = w1x_ref[0].astype(jnp.bfloat16)
                w2b_s[...] = w2x_ref[0].astype(jnp.bfloat16)

            @pl.when(p == 1)
            def _():
                w1b_s[...] = w1y_ref[0].astype(jnp.bfloat16)
                w2b_s[...] = w2y_ref[0].astype(jnp.bfloat16)

        last_s[0] = e
        wv = lax.bitcast_convert_type(x_ref[...], jnp.uint32)
        xlo = lax.bitcast_convert_type(
            (wv & 0xFFFF).astype(jnp.uint16), jnp.bfloat16)
        xhi = lax.bitcast_convert_type(
            (wv >> 16).astype(jnp.uint16), jnp.bfloat16)
        h = (jnp.dot(xlo, w1b_s[:D // 2], preferred_element_type=jnp.float32)
             + jnp.dot(xhi, w1b_s[D // 2:],
                       preferred_element_type=jnp.float32)
             + b1_ref[0])
        hb = jnp.maximum(h, 0.0).astype(jnp.bfloat16)
        y = (jnp.dot(hb, w2b_s[...], preferred_element_type=jnp.float32)
             + b2_ref[0])
        o_ref[...] = y


_ffn_call = pl.pallas_call(
    _ffn_body,
    grid_spec=pltpu.PrefetchScalarGridSpec(
        num_scalar_prefetch=1,
        grid=(G,),
        in_specs=[
            pl.BlockSpec((BM, D // 2), lambda g, s: (_gg(g, s), 0)),
            pl.BlockSpec((1, D, DFF),
                         lambda g, s: (s[0, SX_L + _gg(g, s)], 0, 0)),
            pl.BlockSpec((1, D, DFF),
                         lambda g, s: (s[0, SY_L + _gg(g, s)], 0, 0)),
            pl.BlockSpec((1, 1, DFF), lambda g, s: (s[0, _gg(g, s)], 0, 0)),
            pl.BlockSpec((1, DFF, D),
                         lambda g, s: (s[0, SX_L + _gg(g, s)], 0, 0)),
            pl.BlockSpec((1, DFF, D),
                         lambda g, s: (s[0, SY_L + _gg(g, s)], 0, 0)),
            pl.BlockSpec((1, 1, D), lambda g, s: (s[0, _gg(g, s)], 0, 0)),
        ],
        out_specs=pl.BlockSpec((BM, D), lambda g, s: (_gg(g, s), 0)),
        scratch_shapes=[
            pltpu.VMEM((D, DFF), jnp.bfloat16),
            pltpu.VMEM((DFF, D), jnp.bfloat16),
            pltpu.SMEM((1,), jnp.int32),
        ],
    ),
    out_shape=jax.ShapeDtypeStruct((TP, D), jnp.float32),
    compiler_params=pltpu.CompilerParams(
        dimension_semantics=("arbitrary",)),
)


@functools.partial(
    pl.kernel,
    out_type=jax.ShapeDtypeStruct((T, D), jnp.float32),
    mesh=_sc_mesh,
    scratch_types=[
        pltpu.VMEM((NSUB, SUB), jnp.int32),
        pltpu.VMEM((SUBC, D), jnp.float32),
        pltpu.VMEM((SUBC, D), jnp.float32),
        pltpu.SemaphoreType.DMA,
        pltpu.SemaphoreType.DMA,
    ],
    compiler_params=pltpu.CompilerParams(needs_layout_passes=False),
)
def _combine_kernel(ysp_hbm, pos_hbm, out_hbm, idx_v, rows0, rows1,
                    gsem, wsem):
    w = lax.axis_index("s") * NC + lax.axis_index("c")
    base = w * CH
    rows = (rows0, rows1)
    pltpu.sync_copy(pos_hbm.at[w], idx_v)

    def _idx(j):
        return idx_v.at[j // 2, pl.ds((j % 2) * SUBC, SUBC)]

    gets = [pltpu.async_copy(ysp_hbm.at[_idx(0)], rows0, gsem)]
    wrs = []
    for j in range(NSUBC):
        gets[j].wait()
        wrs.append(pltpu.async_copy(
            rows[j % 2], out_hbm.at[pl.ds(base + j * SUBC, SUBC)], wsem))
        if j + 1 < NSUBC:
            if j >= 1:
                wrs[j - 1].wait()
            gets.append(pltpu.async_copy(
                ysp_hbm.at[_idx(j + 1)], rows[(j + 1) % 2], gsem))
    for j in range(max(NSUBC - 2, 0), NSUBC):
        wrs[j].wait()


def kernel(moe_inp, Wg, bg, w1, b1, w2, b2):
    er3, meta, xb16 = _gate_call(moe_inp, Wg, bg.reshape(1, E))
    xsp, pos = _dispatch_kernel(xb16, er3, meta)
    ysp = _ffn_call(meta, xsp, w1, w1, b1.reshape(E, 1, DFF), w2, w2,
                    b2.reshape(E, 1, D))
    return _combine_kernel(ysp, pos)


# speedup vs baseline: 1.0140x; 1.0140x over previous
import functools

import jax
import jax.numpy as jnp
from jax import lax
from jax.experimental import pallas as pl
from jax.experimental.pallas import tpu as pltpu
from jax.experimental.pallas import tpu_sc as plsc

E = 16
D = 1024
DFF = 1024
T = 8192

BM = 512
NT = T // BM
G = NT + E
TP = G * BM

MW = 256
EGW = 128
OFFP_L = 128
USED_L = 160

RT = 1024
NRT = T // RT

NC = 2
NS = 16
NW = NC * NS
CH = T // NW
NRW = RT // CH
SUB = 64
NSUB = CH // SUB
SUBC = 32
NSUBC = CH // SUBC


def _gate_body(x_ref, wg_ref, bg_ref, er_ref, meta_ref, xb_ref, run_ref):
    t = pl.program_id(0)

    @pl.when(t == 0)
    def _():
        run_ref[...] = jnp.zeros_like(run_ref)

    xb = x_ref[...].astype(jnp.bfloat16)
    lo = lax.bitcast_convert_type(xb[:, :D // 2], jnp.uint16)
    hi = lax.bitcast_convert_type(xb[:, D // 2:], jnp.uint16)
    xb_ref[...] = lax.bitcast_convert_type(
        lo.astype(jnp.uint32) | (hi.astype(jnp.uint32) << 16), jnp.int32)
    logits = jnp.dot(x_ref[...], wg_ref[...],
                     preferred_element_type=jnp.float32) + bg_ref[...]
    m = jnp.max(logits, axis=1, keepdims=True)
    ii = lax.broadcasted_iota(jnp.int32, (RT, E), 1)
    eid = jnp.min(jnp.where(logits == m, ii, E), axis=1)
    oh = (ii == eid[:, None]).astype(jnp.float32)
    ri = lax.broadcasted_iota(jnp.int32, (RT, RT), 0)
    ci = lax.broadcasted_iota(jnp.int32, (RT, RT), 1)
    lt = (ci < ri).astype(jnp.bfloat16)
    within = jnp.dot(lt, oh.astype(jnp.bfloat16),
                     preferred_element_type=jnp.float32)
    run = run_ref[...]
    rank = jnp.sum(oh * (within + run), axis=1)
    er_ref[...] = ((eid << 16) | rank.astype(jnp.int32))[None, None, :]
    new_run = run + jnp.sum(oh, axis=0, keepdims=True)
    run_ref[...] = new_run

    @pl.when(t == NRT - 1)
    def _():
        tiles = jnp.ceil(new_run * (1.0 / BM))
        ut = (lax.broadcasted_iota(jnp.int32, (E, E), 0)
              <= lax.broadcasted_iota(jnp.int32, (E, E), 1)).astype(jnp.bfloat16)
        ends = jnp.dot(tiles.astype(jnp.bfloat16), ut,
                       preferred_element_type=jnp.float32)
        offp = ((ends - tiles) * BM).astype(jnp.int32)
        used = jnp.max(ends, axis=1, keepdims=True).astype(jnp.int32)
        gi = lax.broadcasted_iota(jnp.int32, (EGW, E), 0)
        eg = jnp.minimum(
            jnp.sum((gi >= ends.astype(jnp.int32)).astype(jnp.int32), axis=1),
            E - 1)
        meta_ref[...] = jnp.concatenate(
            [eg[None, :], offp,
             jnp.broadcast_to(used, (1, MW - EGW - E))], axis=1)


_gate_call = pl.pallas_call(
    _gate_body,
    grid=(NRT,),
    in_specs=[
        pl.BlockSpec((RT, D), lambda t: (t, 0)),
        pl.BlockSpec((D, E), lambda t: (0, 0)),
        pl.BlockSpec((1, E), lambda t: (0, 0)),
    ],
    out_specs=[
        pl.BlockSpec((1, 1, RT), lambda t: (t, 0, 0)),
        pl.BlockSpec((1, MW), lambda t: (0, 0)),
        pl.BlockSpec((RT, D // 2), lambda t: (t, 0)),
    ],
    out_shape=[
        jax.ShapeDtypeStruct((NRT, 1, RT), jnp.int32),
        jax.ShapeDtypeStruct((1, MW), jnp.int32),
        jax.ShapeDtypeStruct((T, D // 2), jnp.int32),
    ],
    scratch_shapes=[pltpu.VMEM((1, E), jnp.float32)],
    compiler_params=pltpu.CompilerParams(
        dimension_semantics=("arbitrary",)),
)


_sc_mesh = plsc.VectorSubcoreMesh(core_axis_name="c", subcore_axis_name="s")


@functools.partial(
    pl.kernel,
    out_type=[
        jax.ShapeDtypeStruct((TP, D // 2), jnp.int32),
        jax.ShapeDtypeStruct((NW, NSUB, SUB), jnp.int32),
    ],
    mesh=_sc_mesh,
    scratch_types=[
        pltpu.VMEM((CH,), jnp.int32),
        pltpu.VMEM((E,), jnp.int32),
        pltpu.VMEM((NSUB, SUB), jnp.int32),
        pltpu.VMEM((SUB, D // 2), jnp.int32),
        pltpu.VMEM((SUB, D // 2), jnp.int32),
        pltpu.SemaphoreType.DMA,
        pltpu.SemaphoreType.DMA,
    ],
    compiler_params=pltpu.CompilerParams(needs_layout_passes=False),
)
def _dispatch_kernel(x_hbm, er_hbm, meta_hbm, xsp_hbm, pos_hbm,
                     er_v, off_v, idx_v, rows0, rows1, rsem, ssem):
    w = lax.axis_index("s") * NC + lax.axis_index("c")
    base = w * CH
    r, lo = w // NRW, (w % NRW) * CH
    rows = (rows0, rows1)
    reads = [pltpu.async_copy(x_hbm.at[pl.ds(base, SUB)], rows0, rsem)]
    pltpu.sync_copy(meta_hbm.at[0, pl.ds(OFFP_L, E)], off_v)
    pltpu.sync_copy(er_hbm.at[r, 0, pl.ds(lo, CH)], er_v)
    for j in range(NSUB):
        for k in range(SUB // 16):
            o = j * SUB + k * 16
            ev = er_v[pl.ds(o, 16)]
            idx_v[j, pl.ds(k * 16, 16)] = (
                plsc.load_gather(off_v, [ev >> 16]) + (ev & 0xFFFF))
    scats = []
    for j in range(NSUB):
        reads[j].wait()
        scats.append(
            pltpu.async_copy(rows[j % 2], xsp_hbm.at[idx_v.at[j]], ssem))
        if j + 1 < NSUB:
            if j >= 1:
                scats[j - 1].wait()
            reads.append(pltpu.async_copy(
                x_hbm.at[pl.ds(base + (j + 1) * SUB, SUB)],
                rows[(j + 1) % 2], rsem))
    pltpu.sync_copy(idx_v, pos_hbm.at[w])
    for j in range(max(NSUB - 2, 0), NSUB):
        scats[j].wait()


def _gg(g, s):
    return jnp.minimum(g, s[0, USED_L] - 1)


def _ffn_body(s_ref, x_ref, w1_ref, b1_ref, w2_ref, b2_ref, o_ref,
              w1b_s, w2b_s, last_s):
    g = pl.program_id(0)

    @pl.when(g < s_ref[0, USED_L])
    def _():
        e = s_ref[0, g]

        @pl.when(jnp.logical_or(g == 0, e != last_s[0]))
        def _():
            w1b_s[...] = w1_ref[0].astype(jnp.bfloat16)
            w2b_s[...] = w2_ref[0].astype(jnp.bfloat16)

        last_s[0] = e
        wv = lax.bitcast_convert_type(x_ref[...], jnp.uint32)
        xlo = lax.bitcast_convert_type(
            (wv & 0xFFFF).astype(jnp.uint16), jnp.bfloat16)
        xhi = lax.bitcast_convert_type(
            (wv >> 16).astype(jnp.uint16), jnp.bfloat16)
        h = (jnp.dot(xlo, w1b_s[:D // 2], preferred_element_type=jnp.float32)
             + jnp.dot(xhi, w1b_s[D // 2:],
                       preferred_element_type=jnp.float32)
             + b1_ref[0])
        hb = jnp.maximum(h, 0.0).astype(jnp.bfloat16)
        y = (jnp.dot(hb, w2b_s[...], preferred_element_type=jnp.float32)
             + b2_ref[0])
        o_ref[...] = y


_ffn_call = pl.pallas_call(
    _ffn_body,
    grid_spec=pltpu.PrefetchScalarGridSpec(
        num_scalar_prefetch=1,
        grid=(G,),
        in_specs=[
            pl.BlockSpec((BM, D // 2), lambda g, s: (_gg(g, s), 0)),
            pl.BlockSpec((1, D, DFF), lambda g, s: (s[0, _gg(g, s)], 0, 0)),
            pl.BlockSpec((1, 1, DFF), lambda g, s: (s[0, _gg(g, s)], 0, 0)),
            pl.BlockSpec((1, DFF, D), lambda g, s: (s[0, _gg(g, s)], 0, 0)),
            pl.BlockSpec((1, 1, D), lambda g, s: (s[0, _gg(g, s)], 0, 0)),
        ],
        out_specs=pl.BlockSpec((BM, D), lambda g, s: (_gg(g, s), 0)),
        scratch_shapes=[
            pltpu.VMEM((D, DFF), jnp.bfloat16),
            pltpu.VMEM((DFF, D), jnp.bfloat16),
            pltpu.SMEM((1,), jnp.int32),
        ],
    ),
    out_shape=jax.ShapeDtypeStruct((TP, D), jnp.float32),
    compiler_params=pltpu.CompilerParams(
        dimension_semantics=("arbitrary",)),
)


@functools.partial(
    pl.kernel,
    out_type=jax.ShapeDtypeStruct((T, D), jnp.float32),
    mesh=_sc_mesh,
    scratch_types=[
        pltpu.VMEM((NSUB, SUB), jnp.int32),
        pltpu.VMEM((SUBC, D), jnp.float32),
        pltpu.VMEM((SUBC, D), jnp.float32),
        pltpu.SemaphoreType.DMA,
        pltpu.SemaphoreType.DMA,
    ],
    compiler_params=pltpu.CompilerParams(needs_layout_passes=False),
)
def _combine_kernel(ysp_hbm, pos_hbm, out_hbm, idx_v, rows0, rows1,
                    gsem, wsem):
    w = lax.axis_index("s") * NC + lax.axis_index("c")
    base = w * CH
    rows = (rows0, rows1)
    pltpu.sync_copy(pos_hbm.at[w], idx_v)

    def _idx(j):
        return idx_v.at[j // 2, pl.ds((j % 2) * SUBC, SUBC)]

    gets = [pltpu.async_copy(ysp_hbm.at[_idx(0)], rows0, gsem)]
    wrs = []
    for j in range(NSUBC):
        gets[j].wait()
        wrs.append(pltpu.async_copy(
            rows[j % 2], out_hbm.at[pl.ds(base + j * SUBC, SUBC)], wsem))
        if j + 1 < NSUBC:
            if j >= 1:
                wrs[j - 1].wait()
            gets.append(pltpu.async_copy(
                ysp_hbm.at[_idx(j + 1)], rows[(j + 1) % 2], gsem))
    for j in range(max(NSUBC - 2, 0), NSUBC):
        wrs[j].wait()


def kernel(moe_inp, Wg, bg, w1, b1, w2, b2):
    er3, meta, xb16 = _gate_call(moe_inp, Wg, bg.reshape(1, E))
    xsp, pos = _dispatch_kernel(xb16, er3, meta)
    ysp = _ffn_call(meta, xsp, w1, b1.reshape(E, 1, DFF), w2,
                    b2.reshape(E, 1, D))
    return _combine_kernel(ysp, pos)
